# baseline (speedup 1.0000x reference)
import functools

import jax
import jax.numpy as jnp
from jax import lax
from jax.experimental import pallas as pl
from jax.experimental.pallas import tpu as pltpu
from jax.experimental.pallas import tpu_sc as plsc

N = 10000
D = 128
E = 320000
NPAD = 10240
NSUB = 16
NCORE = 2
NW = NCORE * NSUB
CK = 128
CH = 80
SGC = 16
EP = NW * CH * CK
RPS = NPAD // NSUB
WCH = 128

_mesh = plsc.VectorSubcoreMesh(core_axis_name="c", subcore_axis_name="s")


@functools.partial(
    pl.kernel,
    out_type=jax.ShapeDtypeStruct((NCORE, NPAD, D), jnp.float32),
    mesh=_mesh,
    scratch_types=[
        pltpu.VMEM((SGC, CK), jnp.int32),
        pltpu.VMEM((CK, D), jnp.float32),
        pltpu.VMEM_SHARED((NPAD, D), jnp.float32),
    ],
)
def _deg_kernel(dst_hbm, deg_hbm, dst_v, buf_v, deg_sh):
    c = lax.axis_index("c")
    s = lax.axis_index("s")
    w = c * NSUB + s

    def _zero(i, carry):
        buf_v[i // 8, pl.ds((i % 8) * 16, 16)] = jnp.zeros((16,), jnp.float32)
        return carry

    lax.fori_loop(0, WCH * 8, _zero, 0)

    def _init(k, carry):
        pltpu.sync_copy(buf_v.at[pl.ds(0, WCH)],
                        deg_sh.at[pl.ds(s * RPS + k * WCH, WCH)])
        return carry

    lax.fori_loop(0, RPS // WCH, _init, 0)

    def _ones(i, carry):
        buf_v[i // 8, pl.ds((i % 8) * 16, 16)] = jnp.full((16,), 1.0, jnp.float32)
        return carry

    lax.fori_loop(0, CK * 8, _ones, 0)
    plsc.subcore_barrier()

    def _sg(g, carry):
        pltpu.sync_copy(dst_hbm.at[pl.ds(w * CH + g * SGC, SGC)], dst_v)
        for j in range(SGC):
            pltpu.sync_copy(buf_v, deg_sh.at[dst_v.at[j]], add=True)
        return carry

    lax.fori_loop(0, CH // SGC, _sg, 0)
    plsc.subcore_barrier()

    def _wout(k, carry):
        pltpu.sync_copy(deg_sh.at[pl.ds(s * RPS + k * WCH, WCH)],
                        buf_v.at[pl.ds(0, WCH)])
        pltpu.sync_copy(buf_v.at[pl.ds(0, WCH)],
                        deg_hbm.at[c, pl.ds(s * RPS + k * WCH, WCH)])
        return carry

    lax.fori_loop(0, RPS // WCH, _wout, 0)


@functools.partial(
    pl.kernel,
    out_type=jax.ShapeDtypeStruct((NCORE, NPAD, D), jnp.float32),
    mesh=_mesh,
    scratch_types=[
        pltpu.VMEM((SGC, CK), jnp.int32),
        pltpu.VMEM((SGC, CK), jnp.int32),
        pltpu.VMEM((CK, D), jnp.float32),
        pltpu.VMEM((CK, D), jnp.float32),
        pltpu.VMEM_SHARED((NPAD, D), jnp.float32),
        pltpu.SemaphoreType.DMA,
        pltpu.SemaphoreType.DMA,
        pltpu.SemaphoreType.DMA,
        pltpu.SemaphoreType.DMA,
    ],
)
def _agg_kernel(src_hbm, dst_hbm, xs_hbm, out_hbm,
                src_v, dst_v, rows0, rows1, acc_sh, sem0, sem1, sem2, sem3):
    c = lax.axis_index("c")
    s = lax.axis_index("s")
    w = c * NSUB + s

    def _zero(i, carry):
        rows0[i // 8, pl.ds((i % 8) * 16, 16)] = jnp.zeros((16,), jnp.float32)
        return carry

    lax.fori_loop(0, CK * 8, _zero, 0)

    def _init(k, carry):
        pltpu.sync_copy(rows0, acc_sh.at[pl.ds(s * RPS + k * CK, CK)])
        return carry

    lax.fori_loop(0, RPS // CK, _init, 0)
    plsc.subcore_barrier()

    def _sg(g, carry):
        base = w * CH + g * SGC
        pltpu.sync_copy(src_hbm.at[pl.ds(base, SGC)], src_v)
        pltpu.sync_copy(dst_hbm.at[pl.ds(base, SGC)], dst_v)
        rbuf = (rows0, rows1)
        gsem = (sem0, sem1)
        ssem = (sem2, sem3)
        gd = [pltpu.async_copy(xs_hbm.at[src_v.at[0]], rows0, sem0), None]
        sd = [None, None]
        for j in range(SGC):
            b = j % 2
            o = 1 - b
            if j + 1 < SGC:
                if sd[o] is not None:
                    sd[o].wait()
                    sd[o] = None
                gd[o] = pltpu.async_copy(
                    xs_hbm.at[src_v.at[j + 1]], rbuf[o], gsem[o])
            gd[b].wait()
            sd[b] = pltpu.async_copy(
                rbuf[b], acc_sh.at[dst_v.at[j]], ssem[b], add=True)
        for b in range(2):
            if sd[b] is not None:
                sd[b].wait()
        return carry

    lax.fori_loop(0, CH // SGC, _sg, 0)
    plsc.subcore_barrier()

    def _wout(k, carry):
        pltpu.sync_copy(acc_sh.at[pl.ds(s * RPS + k * CK, CK)], rows0)
        pltpu.sync_copy(rows0, out_hbm.at[c, pl.ds(s * RPS + k * CK, CK)])
        return carry

    lax.fori_loop(0, RPS // CK, _wout, 0)


BLK = 512
GRID = NPAD // BLK

_deg_spec = pl.BlockSpec((NCORE, BLK, D), lambda i: (0, i, 0))
_row_spec = pl.BlockSpec((BLK, D), lambda i: (i, 0))
_par_spec = pl.BlockSpec((NCORE, BLK, D), lambda i: (0, i, 0))
_mat_spec = pl.BlockSpec((D, D), lambda i: (0, 0))
_vec_spec = pl.BlockSpec((1, D), lambda i: (0, 0))
_row_out = jax.ShapeDtypeStruct((NPAD, D), jnp.float32)


def _dinv(deg_ref):
    deg = deg_ref[0, :, 0:1] + deg_ref[1, :, 0:1] + 1.0
    return lax.rsqrt(deg)


def _mm_body(x_ref, w_ref, o_ref):
    o_ref[...] = jnp.dot(x_ref[...], w_ref[...],
                         preferred_element_type=jnp.float32)


_mm_call = pl.pallas_call(
    _mm_body,
    grid=(GRID,),
    in_specs=[_row_spec, _mat_spec],
    out_specs=_row_spec,
    out_shape=_row_out,
)


def _scale_body(deg_ref, xw_ref, o_ref):
    o_ref[...] = _dinv(deg_ref) * xw_ref[...]


_scale_call = pl.pallas_call(
    _scale_body,
    grid=(GRID,),
    in_specs=[_deg_spec, _row_spec],
    out_specs=_row_spec,
    out_shape=_row_out,
)


def _mid_body(deg_ref, p_ref, xs1_ref, w_ref, b_ref, o_ref):
    dinv = _dinv(deg_ref)
    agg = p_ref[0] + p_ref[1] + xs1_ref[...]
    h = jnp.maximum(dinv * agg + b_ref[...], 0.0)
    o_ref[...] = dinv * jnp.dot(h, w_ref[...],
                                preferred_element_type=jnp.float32)


_mid_call = pl.pallas_call(
    _mid_body,
    grid=(GRID,),
    in_specs=[_deg_spec, _par_spec, _row_spec, _mat_spec, _vec_spec],
    out_specs=_row_spec,
    out_shape=_row_out,
)


def _fin_body(deg_ref, q_ref, xs2_ref, b_ref, o_ref):
    dinv = _dinv(deg_ref)
    o_ref[...] = dinv * (q_ref[0] + q_ref[1] + xs2_ref[...]) + b_ref[...]


_fin_call = pl.pallas_call(
    _fin_body,
    grid=(GRID,),
    in_specs=[_deg_spec, _par_spec, _row_spec, _vec_spec],
    out_specs=_row_spec,
    out_shape=_row_out,
)


def kernel(x, edge_index, W1, b1, W2, b2):
    src = edge_index[0]
    dst = edge_index[1]
    pad = EP - E
    srcb = jnp.concatenate(
        [src, jnp.full((pad,), N, jnp.int32)]).reshape(NW * CH, CK)
    dstb = jnp.concatenate(
        [dst, jnp.full((pad,), N, jnp.int32)]).reshape(NW * CH, CK)
    xp = jnp.pad(x, ((0, NPAD - N), (0, 0)))

    degp = _deg_kernel(dstb)
    xw1 = _mm_call(xp, W1)
    xs1 = _scale_call(degp, xw1)
    p = _agg_kernel(srcb, dstb, xs1)
    xs2 = _mid_call(degp, p, xs1, W2, b1.reshape(1, D))
    q = _agg_kernel(srcb, dstb, xs2)
    out = _fin_call(degp, q, xs2, b2.reshape(1, D))
    return out[:N]


# speedup vs baseline: 1.0070x; 1.0070x over previous
import functools

import jax
import jax.numpy as jnp
from jax import lax
from jax.experimental import pallas as pl
from jax.experimental.pallas import tpu as pltpu
from jax.experimental.pallas import tpu_sc as plsc

N = 10000
D = 128
E = 320000
NPAD = 10240
NSUB = 16
NCORE = 2
NW = NCORE * NSUB
CK = 128
CH = 80
SGC = 16
EP = NW * CH * CK
RPS = NPAD // NSUB
WCH = 128

_mesh = plsc.VectorSubcoreMesh(core_axis_name="c", subcore_axis_name="s")


@functools.partial(
    pl.kernel,
    out_type=jax.ShapeDtypeStruct((NCORE, NPAD, D), jnp.float32),
    mesh=_mesh,
    scratch_types=[
        pltpu.VMEM((SGC, CK), jnp.int32),
        pltpu.VMEM((CK, D), jnp.float32),
        pltpu.VMEM_SHARED((NPAD, D), jnp.float32),
    ],
)
def _deg_kernel(dst_hbm, deg_hbm, dst_v, buf_v, deg_sh):
    c = lax.axis_index("c")
    s = lax.axis_index("s")
    w = c * NSUB + s

    def _zero(i, carry):
        buf_v[i // 8, pl.ds((i % 8) * 16, 16)] = jnp.zeros((16,), jnp.float32)
        return carry

    lax.fori_loop(0, WCH * 8, _zero, 0)

    def _init(k, carry):
        pltpu.sync_copy(buf_v.at[pl.ds(0, WCH)],
                        deg_sh.at[pl.ds(s * RPS + k * WCH, WCH)])
        return carry

    lax.fori_loop(0, RPS // WCH, _init, 0)

    def _ones(i, carry):
        buf_v[i // 8, pl.ds((i % 8) * 16, 16)] = jnp.full((16,), 1.0, jnp.float32)
        return carry

    lax.fori_loop(0, CK * 8, _ones, 0)
    plsc.subcore_barrier()

    def _sg(g, carry):
        pltpu.sync_copy(dst_hbm.at[pl.ds(w * CH + g * SGC, SGC)], dst_v)
        for j in range(SGC):
            pltpu.sync_copy(buf_v, deg_sh.at[dst_v.at[j]], add=True)
        return carry

    lax.fori_loop(0, CH // SGC, _sg, 0)
    plsc.subcore_barrier()

    def _wout(k, carry):
        pltpu.sync_copy(deg_sh.at[pl.ds(s * RPS + k * WCH, WCH)],
                        buf_v.at[pl.ds(0, WCH)])
        pltpu.sync_copy(buf_v.at[pl.ds(0, WCH)],
                        deg_hbm.at[c, pl.ds(s * RPS + k * WCH, WCH)])
        return carry

    lax.fori_loop(0, RPS // WCH, _wout, 0)


@functools.partial(
    pl.kernel,
    out_type=jax.ShapeDtypeStruct((NCORE, NPAD, D), jnp.float32),
    mesh=_mesh,
    scratch_types=[
        pltpu.VMEM((SGC, CK), jnp.int32),
        pltpu.VMEM((SGC, CK), jnp.int32),
        pltpu.VMEM((CK, D), jnp.float32),
        pltpu.VMEM((CK, D), jnp.float32),
        pltpu.VMEM_SHARED((NPAD, D), jnp.float32),
        pltpu.SemaphoreType.DMA,
        pltpu.SemaphoreType.DMA,
        pltpu.SemaphoreType.DMA,
        pltpu.SemaphoreType.DMA,
    ],
)
def _agg_kernel(src_hbm, dst_hbm, xs_hbm, out_hbm,
                src_v, dst_v, rows0, rows1, acc_sh, sem0, sem1, sem2, sem3):
    c = lax.axis_index("c")
    s = lax.axis_index("s")
    w = c * NSUB + s

    def _zero(i, carry):
        rows0[i // 8, pl.ds((i % 8) * 16, 16)] = jnp.zeros((16,), jnp.float32)
        return carry

    lax.fori_loop(0, CK * 8, _zero, 0)

    def _init(k, carry):
        pltpu.sync_copy(rows0, acc_sh.at[pl.ds(s * RPS + k * CK, CK)])
        return carry

    lax.fori_loop(0, RPS // CK, _init, 0)
    plsc.subcore_barrier()

    def _sg(g, carry):
        base = w * CH + g * SGC
        pltpu.sync_copy(src_hbm.at[pl.ds(base, SGC)], src_v)
        pltpu.sync_copy(dst_hbm.at[pl.ds(base, SGC)], dst_v)
        rbuf = (rows0, rows1)
        gsem = (sem0, sem1)
        ssem = (sem2, sem3)
        gd = [pltpu.async_copy(xs_hbm.at[src_v.at[0]], rows0, sem0), None]
        sd = [None, None]
        for j in range(SGC):
            b = j % 2
            o = 1 - b
            if j + 1 < SGC:
                if sd[o] is not None:
                    sd[o].wait()
                    sd[o] = None
                gd[o] = pltpu.async_copy(
                    xs_hbm.at[src_v.at[j + 1]], rbuf[o], gsem[o])
            gd[b].wait()
            sd[b] = pltpu.async_copy(
                rbuf[b], acc_sh.at[dst_v.at[j]], ssem[b], add=True)
        for b in range(2):
            if sd[b] is not None:
                sd[b].wait()
        return carry

    lax.fori_loop(0, CH // SGC, _sg, 0)
    plsc.subcore_barrier()

    def _wout(k, carry):
        pltpu.sync_copy(acc_sh.at[pl.ds(s * RPS + k * CK, CK)], rows0)
        pltpu.sync_copy(rows0, out_hbm.at[c, pl.ds(s * RPS + k * CK, CK)])
        return carry

    lax.fori_loop(0, RPS // CK, _wout, 0)


BLK = 512
GRID = NPAD // BLK

_deg_spec = pl.BlockSpec((NCORE, BLK, D), lambda i: (0, i, 0))
_row_spec = pl.BlockSpec((BLK, D), lambda i: (i, 0))
_par_spec = pl.BlockSpec((NCORE, BLK, D), lambda i: (0, i, 0))
_mat_spec = pl.BlockSpec((D, D), lambda i: (0, 0))
_vec_spec = pl.BlockSpec((1, D), lambda i: (0, 0))
_row_out = jax.ShapeDtypeStruct((NPAD, D), jnp.float32)


def _dinv(deg_ref):
    deg = deg_ref[0, :, 0:1] + deg_ref[1, :, 0:1] + 1.0
    return lax.rsqrt(deg)


def _xs1_body(deg_ref, x_ref, w_ref, o_ref):
    o_ref[...] = _dinv(deg_ref) * jnp.dot(
        x_ref[...], w_ref[...], preferred_element_type=jnp.float32)


_xs1_call = pl.pallas_call(
    _xs1_body,
    grid=(GRID,),
    in_specs=[_deg_spec, _row_spec, _mat_spec],
    out_specs=_row_spec,
    out_shape=_row_out,
)


def _mid_body(deg_ref, p_ref, xs1_ref, w_ref, b_ref, o_ref):
    dinv = _dinv(deg_ref)
    agg = p_ref[0] + p_ref[1] + xs1_ref[...]
    h = jnp.maximum(dinv * agg + b_ref[...], 0.0)
    o_ref[...] = dinv * jnp.dot(h, w_ref[...],
                                preferred_element_type=jnp.float32)


_mid_call = pl.pallas_call(
    _mid_body,
    grid=(GRID,),
    in_specs=[_deg_spec, _par_spec, _row_spec, _mat_spec, _vec_spec],
    out_specs=_row_spec,
    out_shape=_row_out,
)


def _fin_body(deg_ref, q_ref, xs2_ref, b_ref, o_ref):
    dinv = _dinv(deg_ref)
    o_ref[...] = dinv * (q_ref[0] + q_ref[1] + xs2_ref[...]) + b_ref[...]


_fin_call = pl.pallas_call(
    _fin_body,
    grid=(GRID,),
    in_specs=[_deg_spec, _par_spec, _row_spec, _vec_spec],
    out_specs=_row_spec,
    out_shape=_row_out,
)


def kernel(x, edge_index, W1, b1, W2, b2):
    src = edge_index[0]
    dst = edge_index[1]
    pad = EP - E
    srcb = jnp.concatenate(
        [src, jnp.full((pad,), N, jnp.int32)]).reshape(NW * CH, CK)
    dstb = jnp.concatenate(
        [dst, jnp.full((pad,), N, jnp.int32)]).reshape(NW * CH, CK)
    xp = jnp.pad(x, ((0, NPAD - N), (0, 0)))

    degp = _deg_kernel(dstb)
    xs1 = _xs1_call(degp, xp, W1)
    p = _agg_kernel(srcb, dstb, xs1)
    xs2 = _mid_call(degp, p, xs1, W2, b1.reshape(1, D))
    q = _agg_kernel(srcb, dstb, xs2)
    out = _fin_call(degp, q, xs2, b2.reshape(1, D))
    return out[:N]
